# Initial kernel scaffold; baseline (speedup 1.0000x reference)
import jax
import jax.numpy as jnp
from jax.experimental import pallas as pl


def kernel(user_emb, entity_emb, edge_index, edge_type, inter_edge, inter_edge_w, relation_emb, W_Q, mess_dropout):
    raise NotImplementedError("write your pallas kernel here")


# speedup vs baseline: 3.1833x; 3.1833x over previous
import functools
import math

import jax
import jax.numpy as jnp
from jax import lax
from jax.experimental import pallas as pl
from jax.experimental.pallas import tpu as pltpu
from jax.experimental.pallas import tpu_sc as plsc

N_ENT = 10000
N_USR = 10000
E = 320000
CH = 128
NH = 2
DK = CH // NH
NRELM1 = 7
NHOPS = 2

NPAD = 10240
ROWS_PER_TILE = NPAD // 16
NW = 32
EDGES_PER_W = E // NW
CHUNK = 40
NCHUNK = EDGES_PER_W // CHUNK
_GROUPS = ((0, 16), (16, 16), (32, 8))

_INV_SQRT_DK = 1.0 / math.sqrt(DK)


_BR = 1000


def _tables_body(emb_ref, wq_ref, rel_ref, p_ref, t_ref, v_ref):
    emb = emb_ref[...]
    p = jnp.dot(emb, wq_ref[...], preferred_element_type=jnp.float32)
    p_ref[...] = p
    for r in range(NRELM1):
        row = rel_ref[r:r + 1, :]
        t_ref[r] = p * row
        v_ref[r] = emb * row


def _tc_tables(emb, wq, rel):
    grid = N_ENT // _BR
    return pl.pallas_call(
        _tables_body,
        grid=(grid,),
        in_specs=[
            pl.BlockSpec((_BR, CH), lambda i: (i, 0)),
            pl.BlockSpec((CH, CH), lambda i: (0, 0)),
            pl.BlockSpec((NRELM1, CH), lambda i: (0, 0)),
        ],
        out_specs=[
            pl.BlockSpec((_BR, CH), lambda i: (i, 0)),
            pl.BlockSpec((NRELM1, _BR, CH), lambda i: (0, i, 0)),
            pl.BlockSpec((NRELM1, _BR, CH), lambda i: (0, i, 0)),
        ],
        out_shape=[
            jax.ShapeDtypeStruct((N_ENT, CH), jnp.float32),
            jax.ShapeDtypeStruct((NRELM1, N_ENT, CH), jnp.float32),
            jax.ShapeDtypeStruct((NRELM1, N_ENT, CH), jnp.float32),
        ],
    )(emb, wq, rel)


def _softmax_norm(num_ref, den_ref, colio):
    num = num_ref[0] + num_ref[1]
    den = den_ref[0] + den_ref[1]
    d0 = den[:, 0:1]
    d1 = den[:, 1:2]
    denb = jnp.where(colio < DK, d0, d1)
    agg = num / (denb + 1e-16)
    nrm = jnp.sqrt(jnp.sum(agg * agg, axis=1, keepdims=True))
    return agg / jnp.maximum(nrm, 1e-12)


def _l2res(acc_ref):
    s = acc_ref[0] + acc_ref[1]
    nrm = jnp.sqrt(jnp.sum(s * s, axis=1, keepdims=True))
    return s / jnp.maximum(nrm, 1e-12)


def _mid_body(num_ref, den_ref, uacc_ref, eres_ref, ures_ref,
              eres_o, ures_o, e1_o):
    colio = lax.broadcasted_iota(jnp.int32, (_BR, CH), 1)
    e1 = _softmax_norm(num_ref, den_ref, colio)
    eres_o[...] = eres_ref[...] + e1
    e1_o[...] = e1
    ures_o[...] = ures_ref[...] + _l2res(uacc_ref)


def _tc_mid(num, den, uacc, eres, ures):
    grid = N_ENT // _BR
    return pl.pallas_call(
        _mid_body,
        grid=(grid,),
        in_specs=[
            pl.BlockSpec((2, _BR, CH), lambda i: (0, i, 0)),
            pl.BlockSpec((2, _BR, CH), lambda i: (0, i, 0)),
            pl.BlockSpec((2, _BR, CH), lambda i: (0, i, 0)),
            pl.BlockSpec((_BR, CH), lambda i: (i, 0)),
            pl.BlockSpec((_BR, CH), lambda i: (i, 0)),
        ],
        out_specs=[
            pl.BlockSpec((_BR, CH), lambda i: (i, 0)),
            pl.BlockSpec((_BR, CH), lambda i: (i, 0)),
            pl.BlockSpec((_BR, CH), lambda i: (i, 0)),
        ],
        out_shape=[
            jax.ShapeDtypeStruct((N_ENT, CH), jnp.float32),
            jax.ShapeDtypeStruct((N_USR, CH), jnp.float32),
            jax.ShapeDtypeStruct((N_ENT, CH), jnp.float32),
        ],
    )(num, den, uacc, eres, ures)


def _fin_body(num_ref, den_ref, uacc_ref, eres_ref, ures_ref, eres_o, ures_o):
    colio = lax.broadcasted_iota(jnp.int32, (_BR, CH), 1)
    e2 = _softmax_norm(num_ref, den_ref, colio)
    eres_o[...] = eres_ref[...] + e2
    ures_o[...] = ures_ref[...] + _l2res(uacc_ref)


def _tc_fin(num, den, uacc, eres, ures):
    grid = N_ENT // _BR
    return pl.pallas_call(
        _fin_body,
        grid=(grid,),
        in_specs=[
            pl.BlockSpec((2, _BR, CH), lambda i: (0, i, 0)),
            pl.BlockSpec((2, _BR, CH), lambda i: (0, i, 0)),
            pl.BlockSpec((2, _BR, CH), lambda i: (0, i, 0)),
            pl.BlockSpec((_BR, CH), lambda i: (i, 0)),
            pl.BlockSpec((_BR, CH), lambda i: (i, 0)),
        ],
        out_specs=[
            pl.BlockSpec((_BR, CH), lambda i: (i, 0)),
            pl.BlockSpec((_BR, CH), lambda i: (i, 0)),
        ],
        out_shape=[
            jax.ShapeDtypeStruct((N_ENT, CH), jnp.float32),
            jax.ShapeDtypeStruct((N_USR, CH), jnp.float32),
        ],
    )(num, den, uacc, eres, ures)


_SC_MESH = plsc.VectorSubcoreMesh(core_axis_name="c", subcore_axis_name="s")


def _zero_rows(buf, iters, width):
    zeros16 = jnp.zeros((16,), jnp.float32)

    def body(i, _):
        for j in range(width // 16):
            buf[i, pl.ds(j * 16, 16)] = zeros16
        return 0

    lax.fori_loop(0, iters, body, 0)


def _hop_body(head_hbm, rt_hbm, p_hbm, t_hbm, v_hbm, u_hbm, t2_hbm, w_hbm,
              ent_hbm, num_out, den_out, uacc_out, w0_out, w1_out,
              hidx, rtidx, qb, tb, vb, a0buf, a1buf, wb0, wb1,
              snum, sem0, sem1, sem2):
    cid = lax.axis_index("c")
    sid = lax.axis_index("s")
    wid = sid * 2 + cid
    base = wid * EDGES_PER_W
    row0 = sid * ROWS_PER_TILE

    iota16 = lax.iota(jnp.int32, 16)
    zeros16 = jnp.zeros((16,), jnp.float32)

    q16 = qb.at[pl.ds(0, 16)]

    _zero_rows(a0buf, 16, 16)
    _zero_rows(a1buf, 16, 16)

    def acc_zero():
        _zero_rows(qb, 16, CH)

        def zbody(kk, _):
            rows = row0 + kk * 16 + iota16
            pltpu.sync_copy(q16, snum.at[rows])
            return 0

        lax.fori_loop(0, ROWS_PER_TILE // 16, zbody, 0)

    def acc_flush(out_ref):
        def fbody(kk, _):
            rows = row0 + kk * 16 + iota16
            hs = pl.ds(row0 + kk * 16, 16)
            pltpu.sync_copy(snum.at[rows], q16)
            pltpu.sync_copy(q16, out_ref.at[cid, hs])
            return 0

        lax.fori_loop(0, ROWS_PER_TILE // 16, fbody, 0)

    acc_zero()
    plsc.subcore_barrier()

    def chunk_body(ci, _):
        off = base + ci * CHUNK
        pltpu.sync_copy(head_hbm.at[pl.ds(off, CHUNK)], hidx)
        pltpu.sync_copy(rt_hbm.at[pl.ds(off, CHUNK)], rtidx)
        c0 = pltpu.async_copy(p_hbm.at[hidx], qb, sem0)
        c1 = pltpu.async_copy(t_hbm.at[rtidx], tb, sem1)
        c2 = pltpu.async_copy(v_hbm.at[rtidx], vb, sem2)
        c0.wait()
        c1.wait()
        c2.wait()

        for gb, glen in _GROUPS:
            for i in range(glen):
                e = gb + i
                acc0 = zeros16
                acc1 = zeros16
                for j in range(4):
                    sl = pl.ds(j * 16, 16)
                    acc0 = acc0 + qb[e, sl] * tb[e, sl]
                for j in range(4, 8):
                    sl = pl.ds(j * 16, 16)
                    acc1 = acc1 + qb[e, sl] * tb[e, sl]
                a0buf[i, pl.ds(0, 16)] = acc0
                a1buf[i, pl.ds(0, 16)] = acc1
            s0 = zeros16
            s1 = zeros16
            for d in range(16):
                col = jnp.full((16,), d, jnp.int32)
                s0 = s0 + plsc.load_gather(a0buf, [iota16, col])
                s1 = s1 + plsc.load_gather(a1buf, [iota16, col])
            w0v = jnp.exp(jnp.minimum(s0 * _INV_SQRT_DK, 60.0))
            w1v = jnp.exp(jnp.minimum(s1 * _INV_SQRT_DK, 60.0))
            wb0[pl.ds(gb, 16)] = w0v
            wb1[pl.ds(gb, 16)] = w1v
            for i in range(glen):
                e = gb + i
                ws0 = jnp.full((16,), w0v[i], jnp.float32)
                ws1 = jnp.full((16,), w1v[i], jnp.float32)
                for j in range(4):
                    sl = pl.ds(j * 16, 16)
                    vb[e, sl] = vb[e, sl] * ws0
                for j in range(4, 8):
                    sl = pl.ds(j * 16, 16)
                    vb[e, sl] = vb[e, sl] * ws1

        pltpu.sync_copy(vb, snum.at[hidx], add=True)
        pltpu.sync_copy(wb0.at[pl.ds(0, CHUNK)], w0_out.at[pl.ds(off, CHUNK)])
        pltpu.sync_copy(wb1.at[pl.ds(0, CHUNK)], w1_out.at[pl.ds(off, CHUNK)])
        return 0

    lax.fori_loop(0, NCHUNK, chunk_body, 0)
    plsc.subcore_barrier()
    acc_flush(num_out)

    acc_zero()
    _zero_rows(vb, CHUNK, CH)
    plsc.subcore_barrier()

    def dchunk_body(ci, _):
        off = base + ci * CHUNK
        pltpu.sync_copy(head_hbm.at[pl.ds(off, CHUNK)], hidx)
        pltpu.sync_copy(w0_out.at[pl.ds(off, CHUNK)], wb0.at[pl.ds(0, CHUNK)])
        pltpu.sync_copy(w1_out.at[pl.ds(off, CHUNK)], wb1.at[pl.ds(0, CHUNK)])

        for gb, glen in _GROUPS:
            w0v = wb0[pl.ds(gb, 16)]
            w1v = wb1[pl.ds(gb, 16)]
            for i in range(glen):
                e = gb + i
                ws0 = jnp.full((16,), w0v[i], jnp.float32)
                ws1 = jnp.full((16,), w1v[i], jnp.float32)
                vb[e, pl.ds(0, 16)] = jnp.where(
                    iota16 == 0, ws0,
                    jnp.where(iota16 == 1, ws1, zeros16))

        pltpu.sync_copy(vb, snum.at[hidx], add=True)
        return 0

    lax.fori_loop(0, NCHUNK, dchunk_body, 0)
    plsc.subcore_barrier()
    acc_flush(den_out)

    acc_zero()
    plsc.subcore_barrier()

    def uchunk_body(ci, _):
        off = base + ci * CHUNK
        pltpu.sync_copy(u_hbm.at[pl.ds(off, CHUNK)], hidx)
        pltpu.sync_copy(t2_hbm.at[pl.ds(off, CHUNK)], rtidx)
        c0 = pltpu.async_copy(w_hbm.at[pl.ds(off, CHUNK)],
                              wb0.at[pl.ds(0, CHUNK)], sem0)
        c1 = pltpu.async_copy(ent_hbm.at[rtidx], vb, sem1)
        c0.wait()
        c1.wait()

        for gb, glen in _GROUPS:
            wv = wb0[pl.ds(gb, 16)]
            for i in range(glen):
                e = gb + i
                ws = jnp.full((16,), wv[i], jnp.float32)
                for j in range(CH // 16):
                    sl = pl.ds(j * 16, 16)
                    vb[e, sl] = vb[e, sl] * ws

        pltpu.sync_copy(vb, snum.at[hidx], add=True)
        return 0

    lax.fori_loop(0, NCHUNK, uchunk_body, 0)
    plsc.subcore_barrier()
    acc_flush(uacc_out)


@functools.partial(
    pl.kernel,
    out_type=(
        jax.ShapeDtypeStruct((2, NPAD, CH), jnp.float32),
        jax.ShapeDtypeStruct((2, NPAD, CH), jnp.float32),
        jax.ShapeDtypeStruct((2, NPAD, CH), jnp.float32),
        jax.ShapeDtypeStruct((E,), jnp.float32),
        jax.ShapeDtypeStruct((E,), jnp.float32),
    ),
    mesh=_SC_MESH,
    compiler_params=pltpu.CompilerParams(needs_layout_passes=False),
    scratch_types=[
        pltpu.VMEM((CHUNK,), jnp.int32),
        pltpu.VMEM((CHUNK,), jnp.int32),
        pltpu.VMEM((CHUNK, CH), jnp.float32),
        pltpu.VMEM((CHUNK, CH), jnp.float32),
        pltpu.VMEM((CHUNK, CH), jnp.float32),
        pltpu.VMEM((16, 17), jnp.float32),
        pltpu.VMEM((16, 17), jnp.float32),
        pltpu.VMEM((CHUNK + 16,), jnp.float32),
        pltpu.VMEM((CHUNK + 16,), jnp.float32),
        pltpu.VMEM_SHARED((NPAD, CH), jnp.float32),
        pltpu.SemaphoreType.DMA,
        pltpu.SemaphoreType.DMA,
        pltpu.SemaphoreType.DMA,
    ],
)
def _sc_hop(head_hbm, rt_hbm, p_hbm, t_hbm, v_hbm, u_hbm, t2_hbm, w_hbm,
            ent_hbm, num_out, den_out, uacc_out, w0_out, w1_out,
            hidx, rtidx, qb, tb, vb, a0buf, a1buf, wb0, wb1,
            snum, sem0, sem1, sem2):
    _hop_body(head_hbm, rt_hbm, p_hbm, t_hbm, v_hbm, u_hbm, t2_hbm, w_hbm,
              ent_hbm, num_out, den_out, uacc_out, w0_out, w1_out,
              hidx, rtidx, qb, tb, vb, a0buf, a1buf, wb0, wb1,
              snum, sem0, sem1, sem2)


def kernel(user_emb, entity_emb, edge_index, edge_type, inter_edge,
           inter_edge_w, relation_emb, W_Q, mess_dropout=0):
    head = edge_index[0]
    tail = edge_index[1]
    r = jnp.remainder(edge_type + (NRELM1 - 1), NRELM1).astype(jnp.int32)
    rt = r * N_ENT + tail
    u_idx = inter_edge[0]
    t2_idx = inter_edge[1]

    def hop(emb_in):
        p, t, v = _tc_tables(emb_in, W_Q, relation_emb)
        tflat = t.reshape(NRELM1 * N_ENT, CH)
        vflat = v.reshape(NRELM1 * N_ENT, CH)
        num, den, uacc, _, _ = _sc_hop(head, rt, p, tflat, vflat,
                                       u_idx, t2_idx, inter_edge_w, emb_in)
        return num[:, :N_ENT], den[:, :N_ENT], uacc[:, :N_USR]

    num1, den1, uacc1 = hop(entity_emb)
    eres1, ures1, e1n = _tc_mid(num1, den1, uacc1, entity_emb, user_emb)

    num2, den2, uacc2 = hop(e1n)
    eres2, ures2 = _tc_fin(num2, den2, uacc2, eres1, ures1)
    return (eres2, ures2)
